# baseline (speedup 1.0000x reference)
import jax
import jax.numpy as jnp
from jax.experimental import pallas as pl
from jax.experimental.pallas import tpu as pltpu


def _make_conv3x3s2_kernel(H):
    def _conv3x3s2_kernel(x_ref, w_ref, b_ref, o_ref):
        _, C, S = x_ref.shape
        Cout = w_ref.shape[-1]
        W = S // H
        Hh, Wh = H // 2, W // 2
        Ho, Wo = Hh, Wh
        dt = jnp.bfloat16

        xb = x_ref[0].astype(dt)
        xt = jnp.transpose(xb)
        xp = xt.reshape(S // 2, 2, C)
        xe = xp[:, 0, :].reshape(Hh, 2, Wh, C)
        xo = xp[:, 1, :].reshape(Hh, 2, Wh, C)

        zrow = jnp.zeros((1, Wo, C), dt)
        zcol = jnp.zeros((Ho, 1, C), dt)
        acc = jnp.zeros((Ho * Wo, Cout), jnp.float32)
        for ky in range(3):
            h0, r = ky // 2, ky % 2
            for kx in range(3):
                w0, p = kx // 2, kx % 2
                src = xe if p == 0 else xo
                lhs = src[:, r, :, :]
                if w0:
                    lhs = jnp.concatenate([lhs[:, 1:, :], zcol], axis=1)
                if h0:
                    lhs = jnp.concatenate([lhs[1:, :, :], zrow], axis=0)
                acc = acc + jnp.dot(lhs.reshape(Ho * Wo, C), w_ref[ky, kx],
                                    preferred_element_type=jnp.float32)
        o_ref[0] = acc + b_ref[...]
    return _conv3x3s2_kernel


def kernel(x_nchw, w_oihw, bias):
    N, C, H, W = x_nchw.shape
    Cout = w_oihw.shape[0]
    Ho = (H - 2) // 2 + 1
    Wo = (W - 2) // 2 + 1

    x = x_nchw.reshape(N, C, H * W)
    wt = jnp.transpose(w_oihw, (2, 3, 1, 0)).astype(jnp.bfloat16)
    b2 = bias.reshape(1, Cout).astype(jnp.float32)

    out = pl.pallas_call(
        _make_conv3x3s2_kernel(H),
        out_shape=jax.ShapeDtypeStruct((N, Ho * Wo, Cout), jnp.float32),
        grid=(N,),
        in_specs=[
            pl.BlockSpec((1, C, H * W), lambda n: (n, 0, 0)),
            pl.BlockSpec((3, 3, C, Cout), lambda n: (0, 0, 0, 0)),
            pl.BlockSpec((1, Cout), lambda n: (0, 0)),
        ],
        out_specs=pl.BlockSpec((1, Ho * Wo, Cout), lambda n: (n, 0, 0)),
        compiler_params=pltpu.CompilerParams(
            dimension_semantics=("parallel",),
            vmem_limit_bytes=96 * 1024 * 1024),
    )(x, wt, b2)

    out = out.reshape(N, Ho, Wo, Cout)
    return jnp.transpose(out, (0, 3, 1, 2))
